# Initial kernel scaffold; baseline (speedup 1.0000x reference)
import jax
import jax.numpy as jnp
from jax.experimental import pallas as pl


def kernel(h_V, h_E, E_idx, W1_w, W1_b, W2_w, W2_b, W3_w, W3_b, Win_w, Win_b, Wout_w, Wout_b):
    raise NotImplementedError("write your pallas kernel here")


# speedup vs baseline: 472.7513x; 472.7513x over previous
import functools

import jax
import jax.numpy as jnp
from jax import lax
from jax.experimental import pallas as pl
from jax.experimental.pallas import tpu as pltpu
from jax.experimental.pallas import tpu_sc as plsc

_NC, _NS = 2, 16
_NW = _NC * _NS
_CH = 128


def _sc_gather(h_V2d, idx_pad, rows_pad, feat):
    chunks_per_w = rows_pad // (_NW * _CH)
    mesh = plsc.VectorSubcoreMesh(core_axis_name="c", subcore_axis_name="s")

    @functools.partial(
        pl.kernel,
        out_type=jax.ShapeDtypeStruct((rows_pad, 128), jnp.float32),
        mesh=mesh,
        scratch_types=[
            pltpu.VMEM((_CH,), jnp.int32),
            pltpu.VMEM((_CH, 128), jnp.float32),
            pltpu.SemaphoreType.DMA,
        ],
    )
    def gather_kernel(table_hbm, idx_hbm, out_hbm, idx_v, rows_v, sem):
        wid = lax.axis_index("s") * _NC + lax.axis_index("c")

        @pl.loop(0, chunks_per_w)
        def _(j):
            base = (wid * chunks_per_w + j) * _CH
            pltpu.sync_copy(idx_hbm.at[pl.ds(base, _CH)], idx_v)
            pltpu.async_copy(table_hbm.at[idx_v], rows_v, sem).wait()
            pltpu.sync_copy(rows_v, out_hbm.at[pl.ds(base, _CH)])

    return gather_kernel(h_V2d, idx_pad)


def _tc_body(gV_ref, gE_ref, hV_ref, W1a_ref, W1b_ref, b1_ref, W2_ref, b2_ref,
             W3_ref, b3_ref, Win_ref, bin_ref, Wout_ref, bout_ref, out_ref,
             *, tile_n, K):
    act = lambda x: 0.5 * x * (1.0 + jax.lax.erf(x * 0.7071067811865476))
    gv = gV_ref[:, :W1a_ref.shape[0]]
    x = gv @ W1a_ref[...] + gE_ref[...] @ W1b_ref[...] + b1_ref[...]
    h = act(x)
    h = act(h @ W2_ref[...] + b2_ref[...])
    m = h @ W3_ref[...] + b3_ref[...]
    dh = jnp.sum(m.reshape(tile_n, K, m.shape[-1]), axis=1) * (1.0 / 30.0)
    hv = hV_ref[...] + dh
    ffn = act(hv @ Win_ref[...] + bin_ref[...]) @ Wout_ref[...] + bout_ref[...]
    out_ref[...] = hv + ffn


def kernel(h_V, h_E, E_idx, W1_w, W1_b, W2_w, W2_b, W3_w, W3_b,
           Win_w, Win_b, Wout_w, Wout_b):
    B, N, H = h_V.shape
    K = h_E.shape[2]
    DE = h_E.shape[3]

    hV2 = h_V.reshape(N, H)
    hE2 = h_E.reshape(N * K, DE)

    rows = N * K
    per_w = _NW * _CH
    rows_pad = ((rows + per_w - 1) // per_w) * per_w
    idx = E_idx.reshape(rows)
    if rows_pad != rows:
        idx = jnp.pad(idx, (0, rows_pad - rows))

    table128 = jnp.pad(hV2, ((0, 0), (0, 128 - H)))
    gV = _sc_gather(table128, idx, rows_pad, H)

    W1a = W1_w[:H]
    W1b = W1_w[H:]

    TILE_N = 400
    grid = (N // TILE_N,)

    full = lambda a: pl.BlockSpec(a.shape, lambda i: (0,) * a.ndim)

    out = pl.pallas_call(
        lambda *refs: _tc_body(*refs, tile_n=TILE_N, K=K),
        grid=grid,
        in_specs=[
            pl.BlockSpec((TILE_N * K, 128), lambda i: (i, 0)),
            pl.BlockSpec((TILE_N * K, DE), lambda i: (i, 0)),
            pl.BlockSpec((TILE_N, H), lambda i: (i, 0)),
            full(W1a), full(W1b), full(W1_b.reshape(1, -1)),
            full(W2_w), full(W2_b.reshape(1, -1)),
            full(W3_w), full(W3_b.reshape(1, -1)),
            full(Win_w), full(Win_b.reshape(1, -1)),
            full(Wout_w), full(Wout_b.reshape(1, -1)),
        ],
        out_specs=pl.BlockSpec((TILE_N, H), lambda i: (i, 0)),
        out_shape=jax.ShapeDtypeStruct((N, H), h_V.dtype),
    )(gV, hE2, hV2, W1a, W1b, W1_b.reshape(1, -1), W2_w, W2_b.reshape(1, -1),
      W3_w, W3_b.reshape(1, -1), Win_w, Win_b.reshape(1, -1),
      Wout_w, Wout_b.reshape(1, -1))

    return out.reshape(B, N, H)
